# baseline (speedup 1.0000x reference)
import functools
import math

import jax
import jax.numpy as jnp
from jax import lax
from jax.experimental import pallas as pl
from jax.experimental.pallas import tpu as pltpu
from jax.experimental.pallas import tpu_sc as plsc

D_MODEL = 64
SCALE = math.sqrt(D_MODEL)

NUM_CORES = 2
NUM_SUBCORES = 16
NW = NUM_CORES * NUM_SUBCORES

CHUNK = 128
NBUF = 5

BV = 4096


def _tr_body(in_ref, out_ref):
    a = in_ref[...] * SCALE
    y = a.T
    z = y.reshape(BV // 2, 2, D_MODEL)
    out_ref[:, 0:D_MODEL] = z[:, 0, :]
    out_ref[:, D_MODEL:2 * D_MODEL] = z[:, 1, :]


def _sc_body(xT_hbm, lut_hbm, out_hbm, idx_v, *scratch):
    rows = scratch[0:NBUF]
    tbufs = scratch[NBUF:2 * NBUF]
    gsems = scratch[2 * NBUF:3 * NBUF]
    osems = scratch[3 * NBUF:4 * NBUF]
    n_s = out_hbm.shape[0]
    wid = lax.axis_index("s") * NUM_CORES + lax.axis_index("c")

    pltpu.sync_copy(xT_hbm.at[:, pl.ds(wid * CHUNK, CHUNK)], idx_v)

    iota = lax.iota(jnp.int32, 16)
    blk_rows = [16 * blk + iota for blk in range(CHUNK // 16)]

    def fire_gather(t, b):
        pltpu.async_copy(lut_hbm.at[idx_v.at[t]], rows[b], gsems[b])

    def wait_gather(b):
        pltpu.make_async_copy(
            lut_hbm.at[idx_v.at[0]], rows[b], gsems[b]).wait()

    def wait_scatter(b):
        pltpu.make_async_copy(
            tbufs[b], out_hbm.at[0, :, wid, :, :], osems[b]).wait()

    for g in range(NBUF - 1):
        fire_gather(g, g)

    def outer(tt, carry):
        for b in range(NBUF):
            t = tt * NBUF + b
            wait_gather(b)

            @plsc.parallel_loop(0, D_MODEL // 8, step=1, unroll=1)
            def _t(ch):
                for cl in range(8):
                    col = iota * 0 + (8 * ch + cl)
                    for blk in range(CHUNK // 16):
                        v = plsc.load_gather(rows[b], [blk_rows[blk], col])
                        tbufs[b][ch, cl, pl.ds(16 * blk, 16)] = v

            pltpu.async_copy(tbufs[b], out_hbm.at[t, :, wid, :, :], osems[b])

            bp = (b - 1) % NBUF

            @pl.when(t == 0)
            def _():
                fire_gather(NBUF - 1, NBUF - 1)

            @pl.when(jnp.logical_and(t >= 1, t <= n_s - NBUF))
            def _():
                wait_scatter(bp)
                fire_gather(t + NBUF - 1, bp)
        return carry

    lax.fori_loop(0, n_s // NBUF, outer, None)
    for b in range(NBUF):
        wait_scatter(b)


@jax.jit
def _emb_call(x, lut):
    b, s = x.shape
    v, d = lut.shape
    nb = b // CHUNK

    lut_t = lut.T
    grid = (v + BV - 1) // BV
    scaled2 = pl.pallas_call(
        _tr_body,
        grid=(grid,),
        in_specs=[pl.BlockSpec((d, BV), lambda j: (0, j))],
        out_specs=pl.BlockSpec((BV // 2, 2 * d), lambda j: (j, 0)),
        out_shape=jax.ShapeDtypeStruct((v // 2, 2 * d), jnp.float32),
    )(lut_t)
    table = scaled2.reshape(v, d)

    xT = x.T
    mesh = plsc.VectorSubcoreMesh(core_axis_name="c", subcore_axis_name="s")
    gather_fn = functools.partial(
        pl.kernel,
        out_type=jax.ShapeDtypeStruct((s, d // 8, nb, 8, CHUNK), jnp.float32),
        mesh=mesh,
        scratch_types=[pltpu.VMEM((s, CHUNK), jnp.int32)]
        + [pltpu.VMEM((CHUNK, d), jnp.float32) for _ in range(NBUF)]
        + [pltpu.VMEM((d // 8, 8, CHUNK), jnp.float32) for _ in range(NBUF)]
        + [pltpu.SemaphoreType.DMA for _ in range(2 * NBUF)],
        compiler_params=pltpu.CompilerParams(
            use_tc_tiling_on_sc=False, needs_layout_passes=False),
    )(_sc_body)
    outT5 = gather_fn(xT, table)
    return outT5.transpose(2, 4, 0, 1, 3).reshape(b, s, d)


def kernel(x, lut):
    return _emb_call(x, lut)


# speedup vs baseline: 1.0741x; 1.0741x over previous
import functools
import math

import jax
import jax.numpy as jnp
from jax import lax
from jax.experimental import pallas as pl
from jax.experimental.pallas import tpu as pltpu
from jax.experimental.pallas import tpu_sc as plsc

D_MODEL = 64
SCALE = math.sqrt(D_MODEL)

NUM_CORES = 2
NUM_SUBCORES = 16
NW = NUM_CORES * NUM_SUBCORES

CHUNK = 128
NBUF = 5

BV = 4096


def _tr_body(in_ref, out_ref):
    a = in_ref[...] * SCALE
    y = a.T
    z = y.reshape(BV // 2, 2, D_MODEL)
    out_ref[:, 0:D_MODEL] = z[:, 0, :]
    out_ref[:, D_MODEL:2 * D_MODEL] = z[:, 1, :]


def _sc_body(xT_hbm, lut_hbm, out_hbm, idx_v, *scratch):
    rows = scratch[0:NBUF]
    tbufs = scratch[NBUF:2 * NBUF]
    gsems = scratch[2 * NBUF:3 * NBUF]
    osems = scratch[3 * NBUF:4 * NBUF]
    n_s = out_hbm.shape[0]
    wid = lax.axis_index("s") * NUM_CORES + lax.axis_index("c")

    pltpu.sync_copy(xT_hbm.at[:, pl.ds(wid * CHUNK, CHUNK)], idx_v)

    iota = lax.iota(jnp.int32, 16)
    blk_rows = [16 * blk + iota for blk in range(CHUNK // 16)]

    def fire_gather(t, b):
        pltpu.async_copy(lut_hbm.at[idx_v.at[t]], rows[b], gsems[b])

    def wait_gather(b):
        pltpu.make_async_copy(
            lut_hbm.at[idx_v.at[0]], rows[b], gsems[b]).wait()

    def wait_scatter(b):
        pltpu.make_async_copy(
            tbufs[b], out_hbm.at[0, :, wid, :, :], osems[b]).wait()

    for g in range(NBUF - 1):
        fire_gather(g, g)

    def outer(tt, carry):
        for b in range(NBUF):
            t = tt * NBUF + b
            wait_gather(b)

            @plsc.parallel_loop(0, D_MODEL, step=1, unroll=8)
            def _t(cc):
                ch = cc >> 3
                cl = cc & 7
                col = iota * 0 + cc
                for blk in range(CHUNK // 16):
                    v = plsc.load_gather(rows[b], [blk_rows[blk], col])
                    tbufs[b][ch, cl, pl.ds(16 * blk, 16)] = v

            pltpu.async_copy(tbufs[b], out_hbm.at[t, :, wid, :, :], osems[b])

            bp = (b - 1) % NBUF

            @pl.when(t == 0)
            def _():
                fire_gather(NBUF - 1, NBUF - 1)

            @pl.when(jnp.logical_and(t >= 1, t <= n_s - NBUF))
            def _():
                wait_scatter(bp)
                fire_gather(t + NBUF - 1, bp)
        return carry

    lax.fori_loop(0, n_s // NBUF, outer, None)
    for b in range(NBUF):
        wait_scatter(b)


@jax.jit
def _emb_call(x, lut):
    b, s = x.shape
    v, d = lut.shape
    nb = b // CHUNK

    lut_t = lut.T
    grid = (v + BV - 1) // BV
    scaled2 = pl.pallas_call(
        _tr_body,
        grid=(grid,),
        in_specs=[pl.BlockSpec((d, BV), lambda j: (0, j))],
        out_specs=pl.BlockSpec((BV // 2, 2 * d), lambda j: (j, 0)),
        out_shape=jax.ShapeDtypeStruct((v // 2, 2 * d), jnp.float32),
    )(lut_t)
    table = scaled2.reshape(v, d)

    xT = x.T
    mesh = plsc.VectorSubcoreMesh(core_axis_name="c", subcore_axis_name="s")
    gather_fn = functools.partial(
        pl.kernel,
        out_type=jax.ShapeDtypeStruct((s, d // 8, nb, 8, CHUNK), jnp.float32),
        mesh=mesh,
        scratch_types=[pltpu.VMEM((s, CHUNK), jnp.int32)]
        + [pltpu.VMEM((CHUNK, d), jnp.float32) for _ in range(NBUF)]
        + [pltpu.VMEM((d // 8, 8, CHUNK), jnp.float32) for _ in range(NBUF)]
        + [pltpu.SemaphoreType.DMA for _ in range(2 * NBUF)],
        compiler_params=pltpu.CompilerParams(
            use_tc_tiling_on_sc=False, needs_layout_passes=False),
    )(_sc_body)
    outT5 = gather_fn(xT, table)
    return outT5.transpose(2, 4, 0, 1, 3).reshape(b, s, d)


def kernel(x, lut):
    return _emb_call(x, lut)


# speedup vs baseline: 1.3850x; 1.2894x over previous
import functools
import math

import jax
import jax.numpy as jnp
from jax import lax
from jax.experimental import pallas as pl
from jax.experimental.pallas import tpu as pltpu
from jax.experimental.pallas import tpu_sc as plsc

D_MODEL = 64
SCALE = math.sqrt(D_MODEL)

NUM_CORES = 2
NUM_SUBCORES = 16
NW = NUM_CORES * NUM_SUBCORES

CHUNK = 128
NBUF = 8

BV = 4096


def _tr_body(in_ref, out_ref):
    a = in_ref[...] * SCALE
    y = a.T
    z = y.reshape(BV // 2, 2, D_MODEL)
    out_ref[:, 0:D_MODEL] = z[:, 0, :]
    out_ref[:, D_MODEL:2 * D_MODEL] = z[:, 1, :]


def _sc_body(xT_hbm, lut_hbm, out_hbm, idx_v, *scratch):
    rows = scratch[0:NBUF]
    gsems = scratch[NBUF:2 * NBUF]
    osems = scratch[2 * NBUF:3 * NBUF]
    n_s = xT_hbm.shape[0]
    wid = lax.axis_index("s") * NUM_CORES + lax.axis_index("c")

    pltpu.sync_copy(xT_hbm.at[:, pl.ds(wid * CHUNK, CHUNK)], idx_v)

    def out_slice(t):
        return out_hbm.at[pl.ds(t * (NW * CHUNK) + wid * CHUNK, CHUNK),
                          pl.ds(0, D_MODEL)]

    def fire_gather(t, b):
        pltpu.async_copy(lut_hbm.at[idx_v.at[t]], rows[b], gsems[b])

    def wait_gather(b):
        pltpu.make_async_copy(
            lut_hbm.at[idx_v.at[0]], rows[b], gsems[b]).wait()

    def wait_scatter(b):
        pltpu.make_async_copy(rows[b], out_slice(0), osems[b]).wait()

    for g in range(NBUF - 1):
        fire_gather(g, g)

    def outer(tt, carry):
        for b in range(NBUF):
            t = tt * NBUF + b
            wait_gather(b)
            pltpu.async_copy(rows[b], out_slice(t), osems[b])

            bp = (b - 1) % NBUF

            @pl.when(t == 0)
            def _():
                fire_gather(NBUF - 1, NBUF - 1)

            @pl.when(jnp.logical_and(t >= 1, t <= n_s - NBUF))
            def _():
                wait_scatter(bp)
                fire_gather(t + NBUF - 1, bp)
        return carry

    lax.fori_loop(0, n_s // NBUF, outer, None)
    for b in range(NBUF):
        wait_scatter(b)


def _perm_body(in_ref, out_ref):
    a = in_ref[:, 0:D_MODEL]
    z = a.reshape(32, CHUNK, D_MODEL)
    zt = jnp.transpose(z, (0, 2, 1))
    z4 = zt.reshape(32, 8, 8, CHUNK)
    out_ref[...] = jnp.transpose(z4, (1, 0, 2, 3)).reshape(
        1, 8, 32, 8, CHUNK)


@jax.jit
def _emb_call(x, lut):
    b, s = x.shape
    v, d = lut.shape
    nb = b // CHUNK

    lut_t = lut.T
    grid = (v + BV - 1) // BV
    scaled2 = pl.pallas_call(
        _tr_body,
        grid=(grid,),
        in_specs=[pl.BlockSpec((d, BV), lambda j: (0, j))],
        out_specs=pl.BlockSpec((BV // 2, 2 * d), lambda j: (j, 0)),
        out_shape=jax.ShapeDtypeStruct((v // 2, 2 * d), jnp.float32),
    )(lut_t)
    table = scaled2.reshape(v, d)

    xT = x.T
    mesh = plsc.VectorSubcoreMesh(core_axis_name="c", subcore_axis_name="s")
    gather_fn = functools.partial(
        pl.kernel,
        out_type=jax.ShapeDtypeStruct((b * s, 2 * d), jnp.float32),
        mesh=mesh,
        scratch_types=[pltpu.VMEM((s, CHUNK), jnp.int32)]
        + [pltpu.VMEM((CHUNK, d), jnp.float32) for _ in range(NBUF)]
        + [pltpu.SemaphoreType.DMA for _ in range(2 * NBUF)],
        compiler_params=pltpu.CompilerParams(use_tc_tiling_on_sc=False),
    )(_sc_body)
    y_p = gather_fn(xT, table)

    outT5 = pl.pallas_call(
        _perm_body,
        grid=(s,),
        in_specs=[pl.BlockSpec((nb * CHUNK, 2 * d), lambda i: (i, 0))],
        out_specs=pl.BlockSpec(
            (1, d // 8, nb, 8, CHUNK), lambda i: (i, 0, 0, 0, 0)),
        out_shape=jax.ShapeDtypeStruct((s, d // 8, nb, 8, CHUNK),
                                       jnp.float32),
    )(y_p)
    return outT5.transpose(2, 4, 0, 1, 3).reshape(b, s, d)


def kernel(x, lut):
    return _emb_call(x, lut)


# speedup vs baseline: 1.5478x; 1.1176x over previous
import functools
import math

import jax
import jax.numpy as jnp
from jax import lax
from jax.experimental import pallas as pl
from jax.experimental.pallas import tpu as pltpu
from jax.experimental.pallas import tpu_sc as plsc

D_MODEL = 64
SCALE = math.sqrt(D_MODEL)

NUM_CORES = 2
NUM_SUBCORES = 16
NW = NUM_CORES * NUM_SUBCORES

CHUNK = 128
NBUF = 8

BV = 8192
SPB = 2


def _tr_body(in_ref, out_ref):
    a = in_ref[...] * SCALE
    y = a.T
    z = y.reshape(BV // 2, 2, D_MODEL)
    out_ref[:, 0:D_MODEL] = z[:, 0, :]
    out_ref[:, D_MODEL:2 * D_MODEL] = z[:, 1, :]


def _sc_body(xT_hbm, lut_hbm, out_hbm, idx_v, *scratch):
    rows = scratch[0:NBUF]
    gsems = scratch[NBUF:2 * NBUF]
    osems = scratch[2 * NBUF:3 * NBUF]
    n_s = xT_hbm.shape[0]
    wid = lax.axis_index("s") * NUM_CORES + lax.axis_index("c")

    pltpu.sync_copy(xT_hbm.at[:, pl.ds(wid * CHUNK, CHUNK)], idx_v)

    def out_slice(t):
        return out_hbm.at[pl.ds(t * (NW * CHUNK) + wid * CHUNK, CHUNK),
                          pl.ds(0, D_MODEL)]

    def fire_gather(t, b):
        pltpu.async_copy(lut_hbm.at[idx_v.at[t]], rows[b], gsems[b])

    def wait_gather(b):
        pltpu.make_async_copy(
            lut_hbm.at[idx_v.at[0]], rows[b], gsems[b]).wait()

    def wait_scatter(b):
        pltpu.make_async_copy(rows[b], out_slice(0), osems[b]).wait()

    for g in range(NBUF - 1):
        fire_gather(g, g)

    def outer(tt, carry):
        for b in range(NBUF):
            t = tt * NBUF + b
            wait_gather(b)
            pltpu.async_copy(rows[b], out_slice(t), osems[b])

            bp = (b - 1) % NBUF

            @pl.when(t == 0)
            def _():
                fire_gather(NBUF - 1, NBUF - 1)

            @pl.when(jnp.logical_and(t >= 1, t <= n_s - NBUF))
            def _():
                wait_scatter(bp)
                fire_gather(t + NBUF - 1, bp)
        return carry

    lax.fori_loop(0, n_s // NBUF, outer, None)
    for b in range(NBUF):
        wait_scatter(b)


def _perm_body(in_ref, out_ref):
    a = in_ref[:, 0:D_MODEL]
    z = a.reshape(SPB * 32, CHUNK, D_MODEL)
    zt = jnp.transpose(z, (0, 2, 1))
    z4 = zt.reshape(SPB, 32, 8, 8, CHUNK)
    out_ref[...] = jnp.transpose(z4, (0, 2, 1, 3, 4)).reshape(
        SPB, 8, 32, 8, CHUNK)


@jax.jit
def _emb_call(x, lut):
    b, s = x.shape
    v, d = lut.shape
    nb = b // CHUNK

    lut_t = lut.T
    grid = (v + BV - 1) // BV
    scaled2 = pl.pallas_call(
        _tr_body,
        grid=(grid,),
        in_specs=[pl.BlockSpec((d, BV), lambda j: (0, j))],
        out_specs=pl.BlockSpec((BV // 2, 2 * d), lambda j: (j, 0)),
        out_shape=jax.ShapeDtypeStruct((v // 2, 2 * d), jnp.float32),
    )(lut_t)
    table = scaled2.reshape(v, d)

    xT = x.T
    mesh = plsc.VectorSubcoreMesh(core_axis_name="c", subcore_axis_name="s")
    gather_fn = functools.partial(
        pl.kernel,
        out_type=jax.ShapeDtypeStruct((b * s, 2 * d), jnp.float32),
        mesh=mesh,
        scratch_types=[pltpu.VMEM((s, CHUNK), jnp.int32)]
        + [pltpu.VMEM((CHUNK, d), jnp.float32) for _ in range(NBUF)]
        + [pltpu.SemaphoreType.DMA for _ in range(2 * NBUF)],
        compiler_params=pltpu.CompilerParams(use_tc_tiling_on_sc=False),
    )(_sc_body)
    y_p = gather_fn(xT, table)

    outT5 = pl.pallas_call(
        _perm_body,
        grid=(s // SPB,),
        in_specs=[pl.BlockSpec(
            (SPB * nb * CHUNK, 2 * d), lambda i: (i, 0))],
        out_specs=pl.BlockSpec(
            (SPB, d // 8, nb, 8, CHUNK), lambda i: (i, 0, 0, 0, 0)),
        out_shape=jax.ShapeDtypeStruct((s, d // 8, nb, 8, CHUNK),
                                       jnp.float32),
    )(y_p)
    return outT5.transpose(2, 4, 0, 1, 3).reshape(b, s, d)


def kernel(x, lut):
    return _emb_call(x, lut)


# speedup vs baseline: 1.5839x; 1.0233x over previous
import functools
import math

import jax
import jax.numpy as jnp
from jax import lax
from jax.experimental import pallas as pl
from jax.experimental.pallas import tpu as pltpu
from jax.experimental.pallas import tpu_sc as plsc

D_MODEL = 64
SCALE = math.sqrt(D_MODEL)

NUM_CORES = 2
NUM_SUBCORES = 16
NW = NUM_CORES * NUM_SUBCORES

CHUNK = 128
NBUF = 8

BV = 16384
SPB = 4


def _tr_body(in_ref, out_ref):
    a = in_ref[...] * SCALE
    y = a.T
    z = y.reshape(BV // 2, 2, D_MODEL)
    out_ref[:, 0:D_MODEL] = z[:, 0, :]
    out_ref[:, D_MODEL:2 * D_MODEL] = z[:, 1, :]


def _sc_body(xT_hbm, lut_hbm, out_hbm, idx_v, *scratch):
    rows = scratch[0:NBUF]
    gsems = scratch[NBUF:2 * NBUF]
    osems = scratch[2 * NBUF:3 * NBUF]
    n_s = xT_hbm.shape[0]
    wid = lax.axis_index("s") * NUM_CORES + lax.axis_index("c")

    pltpu.sync_copy(xT_hbm.at[:, pl.ds(wid * CHUNK, CHUNK)], idx_v)

    def out_slice(t):
        return out_hbm.at[pl.ds(t * (NW * CHUNK) + wid * CHUNK, CHUNK),
                          pl.ds(0, D_MODEL)]

    def fire_gather(t, b):
        pltpu.async_copy(lut_hbm.at[idx_v.at[t]], rows[b], gsems[b])

    def wait_gather(b):
        pltpu.make_async_copy(
            lut_hbm.at[idx_v.at[0]], rows[b], gsems[b]).wait()

    def wait_scatter(b):
        pltpu.make_async_copy(rows[b], out_slice(0), osems[b]).wait()

    for g in range(NBUF - 1):
        fire_gather(g, g)

    def outer(tt, carry):
        for b in range(NBUF):
            t = tt * NBUF + b
            wait_gather(b)
            pltpu.async_copy(rows[b], out_slice(t), osems[b])

            bp = (b - 1) % NBUF

            @pl.when(t == 0)
            def _():
                fire_gather(NBUF - 1, NBUF - 1)

            @pl.when(jnp.logical_and(t >= 1, t <= n_s - NBUF))
            def _():
                wait_scatter(bp)
                fire_gather(t + NBUF - 1, bp)
        return carry

    lax.fori_loop(0, n_s // NBUF, outer, None)
    for b in range(NBUF):
        wait_scatter(b)


def _perm_body(in_ref, out_ref):
    a = in_ref[:, 0:D_MODEL]
    z = a.reshape(SPB * 32, CHUNK, D_MODEL)
    zt = jnp.transpose(z, (0, 2, 1))
    z4 = zt.reshape(SPB, 32, 8, 8, CHUNK)
    out_ref[...] = jnp.transpose(z4, (0, 2, 1, 3, 4)).reshape(
        SPB, 8, 32, 8, CHUNK)


@jax.jit
def _emb_call(x, lut):
    b, s = x.shape
    v, d = lut.shape
    nb = b // CHUNK

    lut_t = lut.T
    grid = (v + BV - 1) // BV
    scaled2 = pl.pallas_call(
        _tr_body,
        grid=(grid,),
        in_specs=[pl.BlockSpec((d, BV), lambda j: (0, j))],
        out_specs=pl.BlockSpec((BV // 2, 2 * d), lambda j: (j, 0)),
        out_shape=jax.ShapeDtypeStruct((v // 2, 2 * d), jnp.float32),
    )(lut_t)
    table = scaled2.reshape(v, d)

    xT = x.T
    mesh = plsc.VectorSubcoreMesh(core_axis_name="c", subcore_axis_name="s")
    gather_fn = functools.partial(
        pl.kernel,
        out_type=jax.ShapeDtypeStruct((b * s, 2 * d), jnp.float32),
        mesh=mesh,
        scratch_types=[pltpu.VMEM((s, CHUNK), jnp.int32)]
        + [pltpu.VMEM((CHUNK, d), jnp.float32) for _ in range(NBUF)]
        + [pltpu.SemaphoreType.DMA for _ in range(2 * NBUF)],
        compiler_params=pltpu.CompilerParams(use_tc_tiling_on_sc=False),
    )(_sc_body)
    y_p = gather_fn(xT, table)

    outT5 = pl.pallas_call(
        _perm_body,
        grid=(s // SPB,),
        in_specs=[pl.BlockSpec(
            (SPB * nb * CHUNK, 2 * d), lambda i: (i, 0))],
        out_specs=pl.BlockSpec(
            (SPB, d // 8, nb, 8, CHUNK), lambda i: (i, 0, 0, 0, 0)),
        out_shape=jax.ShapeDtypeStruct((s, d // 8, nb, 8, CHUNK),
                                       jnp.float32),
    )(y_p)
    return outT5.transpose(2, 4, 0, 1, 3).reshape(b, s, d)


def kernel(x, lut):
    return _emb_call(x, lut)


# speedup vs baseline: 1.5971x; 1.0084x over previous
import functools
import math

import jax
import jax.numpy as jnp
from jax import lax
from jax.experimental import pallas as pl
from jax.experimental.pallas import tpu as pltpu
from jax.experimental.pallas import tpu_sc as plsc

D_MODEL = 64
SCALE = math.sqrt(D_MODEL)

NUM_CORES = 2
NUM_SUBCORES = 16
NW = NUM_CORES * NUM_SUBCORES

CHUNK = 128
NBUF = 8

BV = 16384
SPB = 8


def _tr_body(in_ref, out_ref):
    a = in_ref[...] * SCALE
    y = a.T
    z = y.reshape(BV // 2, 2, D_MODEL)
    out_ref[:, 0:D_MODEL] = z[:, 0, :]
    out_ref[:, D_MODEL:2 * D_MODEL] = z[:, 1, :]


def _sc_body(xT_hbm, lut_hbm, out_hbm, idx_v, *scratch):
    rows = scratch[0:NBUF]
    gsems = scratch[NBUF:2 * NBUF]
    osems = scratch[2 * NBUF:3 * NBUF]
    n_s = xT_hbm.shape[0]
    wid = lax.axis_index("s") * NUM_CORES + lax.axis_index("c")

    pltpu.sync_copy(xT_hbm.at[:, pl.ds(wid * CHUNK, CHUNK)], idx_v)

    def out_slice(t):
        return out_hbm.at[pl.ds(t * (NW * CHUNK) + wid * CHUNK, CHUNK),
                          pl.ds(0, D_MODEL)]

    def fire_gather(t, b):
        pltpu.async_copy(lut_hbm.at[idx_v.at[t]], rows[b], gsems[b])

    def wait_gather(b):
        pltpu.make_async_copy(
            lut_hbm.at[idx_v.at[0]], rows[b], gsems[b]).wait()

    def wait_scatter(b):
        pltpu.make_async_copy(rows[b], out_slice(0), osems[b]).wait()

    for g in range(NBUF - 1):
        fire_gather(g, g)

    def outer(tt, carry):
        for b in range(NBUF):
            t = tt * NBUF + b
            wait_gather(b)
            pltpu.async_copy(rows[b], out_slice(t), osems[b])

            bp = (b - 1) % NBUF

            @pl.when(t == 0)
            def _():
                fire_gather(NBUF - 1, NBUF - 1)

            @pl.when(jnp.logical_and(t >= 1, t <= n_s - NBUF))
            def _():
                wait_scatter(bp)
                fire_gather(t + NBUF - 1, bp)
        return carry

    lax.fori_loop(0, n_s // NBUF, outer, None)
    for b in range(NBUF):
        wait_scatter(b)


def _perm_body(in_ref, out_ref):
    a = in_ref[:, 0:D_MODEL]
    z = a.reshape(SPB * 32, CHUNK, D_MODEL)
    zt = jnp.transpose(z, (0, 2, 1))
    z4 = zt.reshape(SPB, 32, 8, 8, CHUNK)
    out_ref[...] = jnp.transpose(z4, (0, 2, 1, 3, 4)).reshape(
        SPB, 8, 32, 8, CHUNK)


@jax.jit
def _emb_call(x, lut):
    b, s = x.shape
    v, d = lut.shape
    nb = b // CHUNK

    lut_t = lut.T
    grid = (v + BV - 1) // BV
    scaled2 = pl.pallas_call(
        _tr_body,
        grid=(grid,),
        in_specs=[pl.BlockSpec((d, BV), lambda j: (0, j))],
        out_specs=pl.BlockSpec((BV // 2, 2 * d), lambda j: (j, 0)),
        out_shape=jax.ShapeDtypeStruct((v // 2, 2 * d), jnp.float32),
    )(lut_t)
    table = scaled2.reshape(v, d)

    xT = x.T
    mesh = plsc.VectorSubcoreMesh(core_axis_name="c", subcore_axis_name="s")
    gather_fn = functools.partial(
        pl.kernel,
        out_type=jax.ShapeDtypeStruct((b * s, 2 * d), jnp.float32),
        mesh=mesh,
        scratch_types=[pltpu.VMEM((s, CHUNK), jnp.int32)]
        + [pltpu.VMEM((CHUNK, d), jnp.float32) for _ in range(NBUF)]
        + [pltpu.SemaphoreType.DMA for _ in range(2 * NBUF)],
        compiler_params=pltpu.CompilerParams(use_tc_tiling_on_sc=False),
    )(_sc_body)
    y_p = gather_fn(xT, table)

    outT5 = pl.pallas_call(
        _perm_body,
        grid=(s // SPB,),
        in_specs=[pl.BlockSpec(
            (SPB * nb * CHUNK, 2 * d), lambda i: (i, 0))],
        out_specs=pl.BlockSpec(
            (SPB, d // 8, nb, 8, CHUNK), lambda i: (i, 0, 0, 0, 0)),
        out_shape=jax.ShapeDtypeStruct((s, d // 8, nb, 8, CHUNK),
                                       jnp.float32),
    )(y_p)
    return outT5.transpose(2, 4, 0, 1, 3).reshape(b, s, d)


def kernel(x, lut):
    return _emb_call(x, lut)
